# baseline (speedup 1.0000x reference)
import functools

import jax
import jax.numpy as jnp
from jax import lax
from jax.experimental import pallas as pl
from jax.experimental.pallas import tpu as pltpu
from jax.experimental.pallas import tpu_sc as plsc

_VOCAB = 1_000_000
_DIM = 32
_BATCH = 16384
_NC = 2
_NS = 16
_NW = _NC * _NS
_BPW = _BATCH // _NW
_CHUNK = 32
_NCHUNK = _BPW // _CHUNK
_GPC = _CHUNK * _DIM
_WGRAN = _VOCAB // 16


def _glove_body(i_hbm, j_hbm, wi_hbm, wj_hbm, bi_hbm, bj_hbm, out_hbm,
                ii_v, jj_v, offi_v, offj_v, bgi_v, bgj_v,
                gi0, gi1, gj0, gj1, bvi_v, bvj_v, out_v,
                semi0, semi1, semj0, semj1, semb):
    wid = lax.axis_index("s") * _NC + lax.axis_index("c")
    base = wid * _BPW

    pltpu.sync_copy(i_hbm.at[pl.ds(base, _BPW)], ii_v)
    pltpu.sync_copy(j_hbm.at[pl.ds(base, _BPW)], jj_v)

    def offsets(g, carry):
        rI = ii_v[pl.ds(g * 16, 16)] // 16
        rJ = jj_v[pl.ds(g * 16, 16)] // 16
        bgi_v[pl.ds(g * 16, 16)] = rI
        bgj_v[pl.ds(g * 16, 16)] = rJ
        k = g // 2
        g2 = g % 2
        for d in range(_DIM):
            s = k * _GPC + d * _CHUNK + g2 * 16
            offi_v[pl.ds(s, 16)] = rI + d * _WGRAN
            offj_v[pl.ds(s, 16)] = rJ + d * _WGRAN
        return carry

    lax.fori_loop(0, 2 * _NCHUNK, offsets, 0)

    cbi = pltpu.async_copy(bi_hbm.at[bgi_v], bvi_v, semb)
    cbj = pltpu.async_copy(bj_hbm.at[bgj_v], bvj_v, semb)

    gis = (gi0, gi1)
    gjs = (gj0, gj1)
    semis = (semi0, semi1)
    semjs = (semj0, semj1)

    def fire(k):
        s = pl.ds(k * _GPC, _GPC)
        pltpu.async_copy(wi_hbm.at[offi_v.at[s]], gis[k % 2], semis[k % 2])
        pltpu.async_copy(wj_hbm.at[offj_v.at[s]], gjs[k % 2], semjs[k % 2])

    def wait(k):
        pltpu.make_async_copy(wi_hbm.at[offi_v.at[pl.ds(0, _GPC)]],
                              gis[k % 2], semis[k % 2]).wait()
        pltpu.make_async_copy(wj_hbm.at[offj_v.at[pl.ds(0, _GPC)]],
                              gjs[k % 2], semjs[k % 2]).wait()

    lane16 = lax.iota(jnp.int32, 16)
    fire(0)
    cbi.wait()
    cbj.wait()

    for k in range(_NCHUNK):
        if k + 1 < _NCHUNK:
            fire(k + 1)
        wait(k)
        gi, gj = gis[k % 2], gjs[k % 2]

        def group(g2, carry, k=k, gi=gi, gj=gj):
            b0 = k * _CHUNK + g2 * 16
            rowv = g2 * 16 + lane16
            laneI = ii_v[pl.ds(b0, 16)] % 16
            laneJ = jj_v[pl.ds(b0, 16)] % 16
            acc = (plsc.load_gather(bvi_v, [b0 + lane16, laneI])
                   + plsc.load_gather(bvj_v, [b0 + lane16, laneJ]))
            for d in range(_DIM):
                row = rowv + d * _CHUNK
                a = plsc.load_gather(gi, [row, laneI])
                b = plsc.load_gather(gj, [row, laneJ])
                acc = acc + a * b
            out_v[pl.ds(b0, 16)] = acc
            return carry

        lax.fori_loop(0, 2, group, 0)

    pltpu.sync_copy(out_v, out_hbm.at[pl.ds(base, _BPW)])


@jax.jit
def _glove_call(i32, j32, wi_g, wj_g, bi_g, bj_g):
    mesh = plsc.VectorSubcoreMesh(core_axis_name="c", subcore_axis_name="s")
    run = pl.kernel(
        _glove_body,
        mesh=mesh,
        compiler_params=pltpu.CompilerParams(
            needs_layout_passes=False, use_tc_tiling_on_sc=False
        ),
        out_type=jax.ShapeDtypeStruct((_BATCH,), jnp.float32),
        scratch_types=[
            pltpu.VMEM((_BPW,), jnp.int32),
            pltpu.VMEM((_BPW,), jnp.int32),
            pltpu.VMEM((_NCHUNK * _GPC,), jnp.int32),
            pltpu.VMEM((_NCHUNK * _GPC,), jnp.int32),
            pltpu.VMEM((_BPW,), jnp.int32),
            pltpu.VMEM((_BPW,), jnp.int32),
            pltpu.VMEM((_GPC, 16), jnp.float32),
            pltpu.VMEM((_GPC, 16), jnp.float32),
            pltpu.VMEM((_GPC, 16), jnp.float32),
            pltpu.VMEM((_GPC, 16), jnp.float32),
            pltpu.VMEM((_BPW, 16), jnp.float32),
            pltpu.VMEM((_BPW, 16), jnp.float32),
            pltpu.VMEM((_BPW,), jnp.float32),
            pltpu.SemaphoreType.DMA,
            pltpu.SemaphoreType.DMA,
            pltpu.SemaphoreType.DMA,
            pltpu.SemaphoreType.DMA,
            pltpu.SemaphoreType.DMA,
        ],
    )
    return run(i32, j32, wi_g, wj_g, bi_g, bj_g)


def kernel(i, j, wi, wj, bi, bj):
    i32 = i.astype(jnp.int32)
    j32 = j.astype(jnp.int32)
    wi_g = wi.T.reshape(_DIM * _WGRAN, 16)
    wj_g = wj.T.reshape(_DIM * _WGRAN, 16)
    bi_g = bi.reshape(_WGRAN, 16)
    bj_g = bj.reshape(_WGRAN, 16)
    out = _glove_call(i32, j32, wi_g, wj_g, bi_g, bj_g)
    return out.reshape(_BATCH, 1)


# speedup vs baseline: 15.8210x; 15.8210x over previous
import functools

import jax
import jax.numpy as jnp
from jax import lax
from jax.experimental import pallas as pl
from jax.experimental.pallas import tpu as pltpu
from jax.experimental.pallas import tpu_sc as plsc

_VOCAB = 1_000_000
_DIM = 32
_BATCH = 16384
_NC = 2
_NS = 16
_NW = _NC * _NS
_BPW = _BATCH // _NW
_NBUF = 4
_NGRP = _BPW // 16


def _glove_body(i_hbm, j_hbm, wi_hbm, wj_hbm, bi_hbm, bj_hbm, out_hbm,
                ii_v, jj_v, dvec_v, out_v, *scr):
    wib = scr[0:_NBUF]
    wjb = scr[_NBUF:2 * _NBUF]
    bib = scr[2 * _NBUF:3 * _NBUF]
    bjb = scr[3 * _NBUF:4 * _NBUF]
    sems = scr[4 * _NBUF:5 * _NBUF]

    wid = lax.axis_index("s") * _NC + lax.axis_index("c")
    base = wid * _BPW

    pltpu.sync_copy(i_hbm.at[pl.ds(base, _BPW)], ii_v)
    pltpu.sync_copy(j_hbm.at[pl.ds(base, _BPW)], jj_v)

    lane16 = lax.iota(jnp.int32, 16)
    dvec_v[...] = lane16
    bands = dvec_v.at[pl.ds(0, 4)]

    def issue(rI, rJ, n):
        cI = pl.multiple_of((rI // 128) * 128, 128)
        cJ = pl.multiple_of((rJ // 128) * 128, 128)
        bI = pl.multiple_of((rI // 16) * 16, 16)
        bJ = pl.multiple_of((rJ // 16) * 16, 16)
        pltpu.async_copy(wi_hbm.at[bands, :, pl.ds(cI, 128)], wib[n], sems[n])
        pltpu.async_copy(wj_hbm.at[bands, :, pl.ds(cJ, 128)], wjb[n], sems[n])
        pltpu.async_copy(bi_hbm.at[pl.ds(bI, 16)], bib[n], sems[n])
        pltpu.async_copy(bj_hbm.at[pl.ds(bJ, 16)], bjb[n], sems[n])

    def drain(n):
        pltpu.make_async_copy(wi_hbm.at[bands, :, pl.ds(0, 128)], wib[n],
                              sems[n]).wait()
        pltpu.make_async_copy(wj_hbm.at[bands, :, pl.ds(0, 128)], wjb[n],
                              sems[n]).wait()
        pltpu.make_async_copy(bi_hbm.at[pl.ds(0, 16)], bib[n],
                              sems[n]).wait()
        pltpu.make_async_copy(bj_hbm.at[pl.ds(0, 16)], bjb[n],
                              sems[n]).wait()

    band_lo = lane16 // 8
    sub_lo = lane16 % 8
    band_hi = band_lo + 2

    rI0 = ii_v[pl.ds(0, 16)]
    rJ0 = jj_v[pl.ds(0, 16)]
    for n in range(_NBUF):
        issue(rI0[n], rJ0[n], n)

    def outer(g, carry):
        rIm = ii_v[pl.ds(g * 16, 16)]
        rJm = jj_v[pl.ds(g * 16, 16)]
        gn = jnp.minimum(g + 1, _NGRP - 1)
        rIn = ii_v[pl.ds(gn * 16, 16)]
        rJn = jj_v[pl.ds(gn * 16, 16)]
        acc = jnp.full((16,), 0.0, jnp.float32)
        for n in range(16):
            slot = n % _NBUF
            drain(slot)
            zero16 = jnp.full((16,), 0, jnp.int32)
            mI = zero16 + (rIm[n] % 128)
            mJ = zero16 + (rJm[n] % 128)
            a0 = plsc.load_gather(wib[slot], [band_lo, sub_lo, mI])
            a1 = plsc.load_gather(wib[slot], [band_hi, sub_lo, mI])
            b0 = plsc.load_gather(wjb[slot], [band_lo, sub_lo, mJ])
            b1 = plsc.load_gather(wjb[slot], [band_hi, sub_lo, mJ])
            eI = zero16 + (rIm[n] % 16)
            eJ = zero16 + (rJm[n] % 16)
            bv = (jnp.where(lane16 == eI, bib[slot][...], 0.0)
                  + jnp.where(lane16 == eJ, bjb[slot][...], 0.0))
            s = a0 * b0 + a1 * b1 + bv
            dot = lax.reduce_sum(s, axes=(0,))
            acc = jnp.where(lane16 == n, dot, acc)
            if n + _NBUF < 16:
                rIx, rJx = rIm[n + _NBUF], rJm[n + _NBUF]
            else:
                rIx, rJx = rIn[n + _NBUF - 16], rJn[n + _NBUF - 16]
            issue(rIx, rJx, slot)
        out_v[pl.ds(g * 16, 16)] = acc
        return carry

    lax.fori_loop(0, _NGRP, outer, 0)

    for n in range(_NBUF):
        drain(n)

    pltpu.sync_copy(out_v, out_hbm.at[pl.ds(base, _BPW)])


@jax.jit
def _glove_call(i32, j32, wi_t, wj_t, bi_flat, bj_flat):
    mesh = plsc.VectorSubcoreMesh(core_axis_name="c", subcore_axis_name="s")
    run = pl.kernel(
        _glove_body,
        mesh=mesh,
        compiler_params=pltpu.CompilerParams(needs_layout_passes=False),
        out_type=jax.ShapeDtypeStruct((_BATCH,), jnp.float32),
        scratch_types=[
            pltpu.VMEM((_BPW,), jnp.int32),
            pltpu.VMEM((_BPW,), jnp.int32),
            pltpu.VMEM((16,), jnp.int32),
            pltpu.VMEM((_BPW,), jnp.float32),
            *[pltpu.VMEM((4, 8, 128), jnp.float32) for _ in range(2 * _NBUF)],
            *[pltpu.VMEM((16,), jnp.float32) for _ in range(2 * _NBUF)],
            *[pltpu.SemaphoreType.DMA for _ in range(_NBUF)],
        ],
    )
    return run(i32, j32, wi_t, wj_t, bi_flat, bj_flat)


def kernel(i, j, wi, wj, bi, bj):
    i32 = i.astype(jnp.int32)
    j32 = j.astype(jnp.int32)
    wi3 = wi.T.reshape(4, 8, _VOCAB)
    wj3 = wj.T.reshape(4, 8, _VOCAB)
    out = _glove_call(i32, j32, wi3, wj3, bi.reshape(-1), bj.reshape(-1))
    return out.reshape(_BATCH, 1)
